# Initial kernel scaffold; baseline (speedup 1.0000x reference)
import jax
import jax.numpy as jnp
from jax.experimental import pallas as pl


def kernel(graph_embedding, x, edge_index, edge_attr, W1_l, b1_l, W2_l, b2_l, W1_g, b1_g, W2_g, b2_g, struct_att, view_att, view_bias):
    raise NotImplementedError("write your pallas kernel here")


# speedup vs baseline: 1.1548x; 1.1548x over previous
import jax
import jax.numpy as jnp
import numpy as np
from jax.experimental import pallas as pl

NEG_SLOPE = 0.2
LAMB = 1.0


def _xp_kernel(xperm_ref, sperm_ref, sa_ref, xp_ref, t_ref):
    xp = xperm_ref[...] * sperm_ref[...]
    xp_ref[...] = xp
    t_ref[...] = jnp.dot(xp, sa_ref[...], preferred_element_type=jnp.float32)


def _sparsemax_kernel(adj_ref, p_ref):
    z = adj_ref[...]
    m = jnp.max(z, axis=1, keepdims=True)
    lo = m - 1.0
    hi = m
    for _ in range(30):
        mid = 0.5 * (lo + hi)
        s = jnp.sum(jnp.maximum(z - mid, 0.0), axis=1, keepdims=True)
        gt = s > 1.0
        lo = jnp.where(gt, mid, lo)
        hi = jnp.where(gt, hi, mid)
    tau0 = 0.5 * (lo + hi)
    sel = z > tau0
    cnt = jnp.sum(jnp.where(sel, 1.0, 0.0), axis=1, keepdims=True)
    ssum = jnp.sum(jnp.where(sel, z, 0.0), axis=1, keepdims=True)
    tau = (ssum - 1.0) / jnp.maximum(cnt, 1.0)
    p_ref[...] = jnp.maximum(z - tau, 0.0)[:, :p_ref.shape[1]]


def kernel(graph_embedding, x, edge_index, edge_attr,
           W1_l, b1_l, W2_l, b2_l, W1_g, b1_g, W2_g, b2_g,
           struct_att, view_att, view_bias):
    N, H = x.shape
    K = int(np.ceil(0.5 * N))
    KP = ((K + 127) // 128) * 128
    f32 = x.dtype
    src = edge_index[0]
    dst = edge_index[1]

    wsum = jnp.zeros((N,), x.dtype).at[dst].add(edge_attr)
    agg = jnp.zeros_like(x).at[dst].add(x[src] * edge_attr[:, None])
    sub_emb = agg / jnp.clip(wsum, 1e-6, None)[:, None]

    def disc_l(a, b):
        z = jnp.concatenate([a, b], axis=1)
        hdd = jax.nn.leaky_relu(z @ W1_l + b1_l, NEG_SLOPE)
        return hdd @ W2_l + b2_l

    pos_l = disc_l(sub_emb, x)
    neg_l = disc_l(sub_emb, jnp.roll(x, 1, axis=0))
    loc_loss = jnp.mean(jax.nn.softplus(-pos_l)) + jnp.mean(jax.nn.softplus(neg_l))
    scores1 = pos_l

    deg = jnp.zeros((N,), x.dtype).at[dst].add(1.0)
    nb = (jnp.zeros_like(x).at[dst].add(x[src])) / jnp.clip(deg, 1.0, None)[:, None]
    g = jnp.broadcast_to(jnp.mean(x, axis=0, keepdims=True), (N, H))

    def disc_g(a):
        z = jnp.concatenate([a, nb, g], axis=1)
        hdd = jax.nn.leaky_relu(z @ W1_g + b1_g, NEG_SLOPE)
        return hdd @ W2_g + b2_g

    pos_g = disc_g(x)
    neg_g = disc_g(jnp.roll(x, 1, axis=0))
    glo_loss = jnp.mean(jax.nn.softplus(-pos_g)) + jnp.mean(jax.nn.softplus(neg_g))
    scores2 = pos_g

    scores_cat = jnp.concatenate([scores1, scores2], axis=1)
    sw = jax.nn.sigmoid(scores_cat @ view_att + view_bias)
    sw = jax.nn.softmax(sw, axis=1)
    scores = jax.nn.sigmoid(jnp.sum(scores_cat * sw, axis=1))

    _, perm = jax.lax.top_k(scores, K)
    x_perm = x[perm]
    s_perm = scores[perm][:, None]
    sa = jnp.stack([struct_att[0, :H], struct_att[0, H:]], axis=1)
    x_p, t = pl.pallas_call(
        _xp_kernel,
        out_shape=[jax.ShapeDtypeStruct((K, H), f32),
                   jax.ShapeDtypeStruct((K, 2), f32)],
    )(x_perm, s_perm, sa)

    node_map = jnp.full((N,), -1, dtype=jnp.int32).at[perm].set(
        jnp.arange(K, dtype=jnp.int32))
    r = node_map[src]
    c = node_map[dst]
    valid = (r >= 0) & (c >= 0)
    arK = jnp.arange(K, dtype=jnp.int32)
    row = jnp.concatenate([jnp.where(valid, r, K), arK])
    col = jnp.concatenate([jnp.where(valid, c, K), arK])
    attr = jnp.concatenate([edge_attr, jnp.zeros((K,), f32)])
    safe_row = jnp.minimum(row, K - 1)
    safe_col = jnp.minimum(col, K - 1)
    w = jax.nn.leaky_relu(t[safe_row, 0] + t[safe_col, 1], NEG_SLOPE) + attr * LAMB
    adj = jnp.full((K, KP), -1e9, f32).at[row, col].set(w, mode='drop')

    R = 40
    P = pl.pallas_call(
        _sparsemax_kernel,
        grid=(K // R,),
        in_specs=[pl.BlockSpec((R, KP), lambda i: (i, 0))],
        out_specs=pl.BlockSpec((R, K), lambda i: (i, 0)),
        out_shape=jax.ShapeDtypeStruct((K, K), f32),
    )(adj)

    return (x_p, P, perm, loc_loss, glo_loss)


# speedup vs baseline: 1.1758x; 1.0182x over previous
import jax
import jax.numpy as jnp
import numpy as np
from jax.experimental import pallas as pl

NEG_SLOPE = 0.2
LAMB = 1.0


def _xp_kernel(xperm_ref, sperm_ref, sa_ref, xp_ref, t_ref):
    xp = xperm_ref[...] * sperm_ref[...]
    xp_ref[...] = xp
    t_ref[...] = jnp.dot(xp, sa_ref[...], preferred_element_type=jnp.float32)


def _sparsemax_kernel(adj_ref, p_ref):
    z = adj_ref[...]
    m = jnp.max(z, axis=1, keepdims=True)
    lo = m - 1.0
    hi = m
    for _ in range(20):
        mid = 0.5 * (lo + hi)
        s = jnp.sum(jnp.maximum(z - mid, 0.0), axis=1, keepdims=True)
        gt = s > 1.0
        lo = jnp.where(gt, mid, lo)
        hi = jnp.where(gt, hi, mid)
    tau0 = 0.5 * (lo + hi)
    sel = z > tau0
    cnt = jnp.sum(jnp.where(sel, 1.0, 0.0), axis=1, keepdims=True)
    ssum = jnp.sum(jnp.where(sel, z, 0.0), axis=1, keepdims=True)
    tau = (ssum - 1.0) / jnp.maximum(cnt, 1.0)
    p_ref[...] = jnp.maximum(z - tau, 0.0)[:, :p_ref.shape[1]]


def kernel(graph_embedding, x, edge_index, edge_attr,
           W1_l, b1_l, W2_l, b2_l, W1_g, b1_g, W2_g, b2_g,
           struct_att, view_att, view_bias):
    N, H = x.shape
    K = int(np.ceil(0.5 * N))
    KP = ((K + 127) // 128) * 128
    f32 = x.dtype
    src = edge_index[0]
    dst = edge_index[1]

    wsum = jnp.zeros((N,), x.dtype).at[dst].add(edge_attr)
    agg = jnp.zeros_like(x).at[dst].add(x[src] * edge_attr[:, None])
    sub_emb = agg / jnp.clip(wsum, 1e-6, None)[:, None]

    def disc_l(a, b):
        z = jnp.concatenate([a, b], axis=1)
        hdd = jax.nn.leaky_relu(z @ W1_l + b1_l, NEG_SLOPE)
        return hdd @ W2_l + b2_l

    pos_l = disc_l(sub_emb, x)
    neg_l = disc_l(sub_emb, jnp.roll(x, 1, axis=0))
    loc_loss = jnp.mean(jax.nn.softplus(-pos_l)) + jnp.mean(jax.nn.softplus(neg_l))
    scores1 = pos_l

    deg = jnp.zeros((N,), x.dtype).at[dst].add(1.0)
    nb = (jnp.zeros_like(x).at[dst].add(x[src])) / jnp.clip(deg, 1.0, None)[:, None]
    g = jnp.broadcast_to(jnp.mean(x, axis=0, keepdims=True), (N, H))

    def disc_g(a):
        z = jnp.concatenate([a, nb, g], axis=1)
        hdd = jax.nn.leaky_relu(z @ W1_g + b1_g, NEG_SLOPE)
        return hdd @ W2_g + b2_g

    pos_g = disc_g(x)
    neg_g = disc_g(jnp.roll(x, 1, axis=0))
    glo_loss = jnp.mean(jax.nn.softplus(-pos_g)) + jnp.mean(jax.nn.softplus(neg_g))
    scores2 = pos_g

    scores_cat = jnp.concatenate([scores1, scores2], axis=1)
    sw = jax.nn.sigmoid(scores_cat @ view_att + view_bias)
    sw = jax.nn.softmax(sw, axis=1)
    scores = jax.nn.sigmoid(jnp.sum(scores_cat * sw, axis=1))

    _, perm = jax.lax.top_k(scores, K)
    x_perm = x[perm]
    s_perm = scores[perm][:, None]
    sa = jnp.stack([struct_att[0, :H], struct_att[0, H:]], axis=1)
    x_p, t = pl.pallas_call(
        _xp_kernel,
        out_shape=[jax.ShapeDtypeStruct((K, H), f32),
                   jax.ShapeDtypeStruct((K, 2), f32)],
    )(x_perm, s_perm, sa)

    node_map = jnp.full((N,), -1, dtype=jnp.int32).at[perm].set(
        jnp.arange(K, dtype=jnp.int32))
    r = node_map[src]
    c = node_map[dst]
    valid = (r >= 0) & (c >= 0)
    arK = jnp.arange(K, dtype=jnp.int32)
    row = jnp.concatenate([jnp.where(valid, r, K), arK])
    col = jnp.concatenate([jnp.where(valid, c, K), arK])
    attr = jnp.concatenate([edge_attr, jnp.zeros((K,), f32)])
    safe_row = jnp.minimum(row, K - 1)
    safe_col = jnp.minimum(col, K - 1)
    w = jax.nn.leaky_relu(t[safe_row, 0] + t[safe_col, 1], NEG_SLOPE) + attr * LAMB
    adj = jnp.full((K, KP), -1e9, f32).at[row, col].set(w, mode='drop')

    R = 200
    P = pl.pallas_call(
        _sparsemax_kernel,
        grid=(K // R,),
        in_specs=[pl.BlockSpec((R, KP), lambda i: (i, 0))],
        out_specs=pl.BlockSpec((R, K), lambda i: (i, 0)),
        out_shape=jax.ShapeDtypeStruct((K, K), f32),
    )(adj)

    return (x_p, P, perm, loc_loss, glo_loss)
